# Initial kernel scaffold; baseline (speedup 1.0000x reference)
import jax
import jax.numpy as jnp
from jax.experimental import pallas as pl


def kernel(features, predictions, targets):
    raise NotImplementedError("write your pallas kernel here")


# speedup vs baseline: 2.4832x; 2.4832x over previous
import jax
import jax.numpy as jnp
from jax.experimental import pallas as pl
from jax.experimental.pallas import tpu as pltpu

_B, _C, _H, _W, _D = 2, 64, 32, 32, 32
_WD = _W * _D
_HT = 8
_NT = _H // _HT
_NPIX = _H * _WD
_MARGIN = 1.0
_W_COMP, _W_SEP, _W_SMOOTH = 1.0, 0.5, 0.3

_NSLOT = 10


def _scl_kernel(f_ref, p_ref, t_ref, out_ref, sums, smem, cf, cc):
    s = pl.program_id(0)
    b = pl.program_id(1)
    i = pl.program_id(2)

    @pl.when((s == 0) & (b == 0) & (i == 0))
    def _init():
        sums[...] = jnp.zeros_like(sums)
        for k in range(_NSLOT):
            smem[k] = 0.0

    f = f_ref[0]
    x0 = p_ref[0, 0]
    x1 = p_ref[0, 1]
    p1 = jax.nn.sigmoid(x1 - x0)
    conf = jnp.maximum(p1, 1.0 - p1)
    t = t_ref[0]
    m1 = (t == 1).astype(jnp.float32)

    @pl.when(s == 0)
    def _pass0():
        fs = jnp.sum(f, axis=(1, 2))
        s1 = jnp.sum(f * m1[None], axis=(1, 2))
        r = 2 * b
        sums[pl.ds(r, 1), :] = sums[pl.ds(r, 1), :] + (fs - s1)[None]
        sums[pl.ds(r + 1, 1), :] = sums[pl.ds(r + 1, 1), :] + s1[None]
        smem[b] = smem[b] + jnp.sum(m1)

        dh = f[:, 1:, :] - f[:, :-1, :]
        sqh = jnp.sum(dh * dh, axis=0)
        wh = (conf[1:] + conf[:-1]) * 0.5
        acc_h = jnp.sum(sqh * wh)

        @pl.when(i > 0)
        def _boundary():
            df = f[:, 0, :] - cf[...]
            sqb = jnp.sum(df * df, axis=0)
            wb = (conf[0] + cc[0]) * 0.5
            smem[6] = smem[6] + jnp.sum(sqb * wb)

        smem[6] = smem[6] + acc_h
        cf[...] = f[:, _HT - 1, :]
        cc[0] = conf[_HT - 1]

        dw = f[:, :, _D:] - f[:, :, :-_D]
        sqw = jnp.sum(dw * dw, axis=0)
        ww = (conf[:, _D:] + conf[:, :-_D]) * 0.5
        smem[7] = smem[7] + jnp.sum(sqw * ww)

        dd = f[:, :, 1:] - f[:, :, :-1]
        sqd = jnp.sum(dd * dd, axis=0)
        lane = jax.lax.broadcasted_iota(jnp.int32, (_HT, _WD - 1), 1)
        valid = (lane % _D) != (_D - 1)
        smem[8] = smem[8] + jnp.sum(jnp.where(valid, sqd, 0.0))

    @pl.when(s == 1)
    def _pass1():
        n1 = smem[b]
        n0 = jnp.float32(_NPIX) - n1
        r = 2 * b
        c0 = sums[pl.ds(r, 1), :][0] / n0
        c1 = sums[pl.ds(r + 1, 1), :][0] / n1
        cc0 = jnp.sum(c0 * c0)
        cc1 = jnp.sum(c1 * c1)

        sq = jnp.sum(f * f, axis=0)
        d0 = jnp.sum(f * c0[:, None, None], axis=0)
        d1 = jnp.sum(f * c1[:, None, None], axis=0)
        dist0 = jnp.sqrt(jnp.maximum(sq - 2.0 * d0 + cc0, 0.0))
        dist1 = jnp.sqrt(jnp.maximum(sq - 2.0 * d1 + cc1, 0.0))
        smem[2 + r] = smem[2 + r] + jnp.sum((1.0 - m1) * dist0 * (1.0 - p1))
        smem[3 + r] = smem[3 + r] + jnp.sum(m1 * dist1 * p1)

        @pl.when(i == 0)
        def _sep():
            dc = c0 - c1
            d01 = jnp.sqrt(jnp.sum(dc * dc))
            smem[9] = smem[9] + jnp.maximum(_MARGIN - d01, 0.0)

    @pl.when((s == 1) & (b == _B - 1) & (i == _NT - 1))
    def _finish():
        comp = jnp.float32(0.0)
        for bb in range(_B):
            n1b = smem[bb]
            n0b = jnp.float32(_NPIX) - n1b
            comp = comp + smem[2 + 2 * bb] / n0b + smem[3 + 2 * bb] / n1b
        comp = comp / jnp.float32(_B * 2)
        sep = smem[9] / jnp.float32(_B)
        denom_hw = jnp.float32(_B * (_H - 1) * _W * _D)
        denom_d = jnp.float32(_B * _C * _H * _W * (_D - 1))
        smooth = smem[6] / denom_hw + smem[7] / denom_hw + 0.1 * smem[8] / denom_d
        out_ref[0, 0] = _W_COMP * comp + _W_SEP * sep + _W_SMOOTH * smooth


@jax.jit
def _run(f, p, t):
    return pl.pallas_call(
        _scl_kernel,
        grid=(2, _B, _NT),
        in_specs=[
            pl.BlockSpec((1, _C, _HT, _WD), lambda s, b, i: (b, 0, i, 0)),
            pl.BlockSpec((1, 2, _HT, _WD), lambda s, b, i: (b, 0, i, 0)),
            pl.BlockSpec((1, _HT, _WD), lambda s, b, i: (b, i, 0)),
        ],
        out_specs=pl.BlockSpec(memory_space=pltpu.SMEM),
        out_shape=jax.ShapeDtypeStruct((1, 1), jnp.float32),
        scratch_shapes=[
            pltpu.VMEM((2 * _B, _C), jnp.float32),
            pltpu.SMEM((_NSLOT,), jnp.float32),
            pltpu.VMEM((_C, _WD), jnp.float32),
            pltpu.VMEM((1, _WD), jnp.float32),
        ],
    )(f, p, t)


def kernel(features, predictions, targets):
    f = features.reshape(_B, _C, _H, _WD)
    p = predictions.reshape(_B, 2, _H, _WD)
    t = targets.astype(jnp.int32).reshape(_B, _H, _WD)
    return _run(f, p, t)[0, 0]
